# Initial kernel scaffold; baseline (speedup 1.0000x reference)
import jax
import jax.numpy as jnp
from jax.experimental import pallas as pl


def kernel(t_e2, h, edge_index1, e1_to_e2, W1, b1, W2, b2):
    raise NotImplementedError("write your pallas kernel here")


# speedup vs baseline: 2.2278x; 2.2278x over previous
import functools

import jax
import jax.numpy as jnp
from jax import lax
from jax.experimental import pallas as pl
from jax.experimental.pallas import tpu as pltpu
from jax.experimental.pallas import tpu_sc as plsc

N_NODES = 10000
E1 = 320000
E2 = 640000
H = 128

NC = 2
NS = 16
NW = NC * NS
EPW = E1 // NW
R = E2 // NW

_mesh = plsc.VectorSubcoreMesh(core_axis_name="c", subcore_axis_name="s")


def _worker_id():
    return lax.axis_index("s") * NC + lax.axis_index("c")


CG = 80
NCH = EPW // CG


def _gather_body(h_hbm, te_hbm, src_hbm, dst_hbm, e2_hbm,
                 hs_hbm, hd_hbm, st_hbm,
                 isrc, idst, ie2, bs, bd, bt, sem0, sem1, sem2):
    base = _worker_id() * EPW

    def chunk(k, carry):
        off = base + k * CG
        pltpu.sync_copy(src_hbm.at[pl.ds(off, CG)], isrc)
        pltpu.sync_copy(dst_hbm.at[pl.ds(off, CG)], idst)
        pltpu.sync_copy(e2_hbm.at[pl.ds(off, CG)], ie2)
        c0 = pltpu.async_copy(h_hbm.at[isrc], bs, sem0)
        c1 = pltpu.async_copy(h_hbm.at[idst], bd, sem1)
        c2 = pltpu.async_copy(te_hbm.at[ie2], bt, sem2)
        c0.wait()
        c1.wait()
        c2.wait()
        pltpu.sync_copy(bs, hs_hbm.at[pl.ds(off, CG)])
        pltpu.sync_copy(bd, hd_hbm.at[pl.ds(off, CG)])
        pltpu.sync_copy(bt, st_hbm.at[pl.ds(off, CG)])
        return carry

    lax.fori_loop(0, NCH, chunk, 0)


_gather_call = pl.kernel(
    _gather_body,
    out_type=(
        jax.ShapeDtypeStruct((E1, H), jnp.float32),
        jax.ShapeDtypeStruct((E1, H), jnp.float32),
        jax.ShapeDtypeStruct((E1, H), jnp.float32),
    ),
    mesh=_mesh,
    scratch_types=[
        pltpu.VMEM((CG,), jnp.int32),
        pltpu.VMEM((CG,), jnp.int32),
        pltpu.VMEM((CG,), jnp.int32),
        pltpu.VMEM((CG, H), jnp.float32),
        pltpu.VMEM((CG, H), jnp.float32),
        pltpu.VMEM((CG, H), jnp.float32),
        pltpu.SemaphoreType.DMA,
        pltpu.SemaphoreType.DMA,
        pltpu.SemaphoreType.DMA,
    ],
    compiler_params=pltpu.CompilerParams(needs_layout_passes=False),
)


BLK = 3200
NB_MLP = E1 // BLK
NB_CPY = E2 // BLK
GRID = NB_MLP + NB_CPY


def _mlp_body(hs, hd, st, te, w1a, w1b, w1c, w2, b1, b2, out):
    g = pl.program_id(0)

    @pl.when(g < NB_MLP)
    def _compute():
        x = (jnp.dot(hs[...], w1a[...], precision=lax.Precision.HIGHEST,
                     preferred_element_type=jnp.float32)
             + jnp.dot(hd[...], w1b[...], precision=lax.Precision.HIGHEST,
                       preferred_element_type=jnp.float32)
             + jnp.dot(st[...], w1c[...], precision=lax.Precision.HIGHEST,
                       preferred_element_type=jnp.float32)
             + b1[...])
        hid = x * jax.nn.sigmoid(x)
        out[...] = (st[...] + b2[...]
                    + jnp.dot(hid, w2[...], precision=lax.Precision.HIGHEST,
                              preferred_element_type=jnp.float32))

    @pl.when(g >= NB_MLP)
    def _copy():
        out[...] = te[...]


def _mlp_vt(hs, hd, st, te, w1a, w1b, w1c, w2, b1, b2):
    return pl.pallas_call(
        _mlp_body,
        grid=(GRID,),
        in_specs=[
            pl.BlockSpec((BLK, H), lambda g: (jnp.minimum(g, NB_MLP - 1), 0)),
            pl.BlockSpec((BLK, H), lambda g: (jnp.minimum(g, NB_MLP - 1), 0)),
            pl.BlockSpec((BLK, H), lambda g: (jnp.minimum(g, NB_MLP - 1), 0)),
            pl.BlockSpec((BLK, H), lambda g: (jnp.maximum(g - NB_MLP, 0), 0)),
            pl.BlockSpec((H, H), lambda g: (0, 0)),
            pl.BlockSpec((H, H), lambda g: (0, 0)),
            pl.BlockSpec((H, H), lambda g: (0, 0)),
            pl.BlockSpec((H, H), lambda g: (0, 0)),
            pl.BlockSpec((1, H), lambda g: (0, 0)),
            pl.BlockSpec((1, H), lambda g: (0, 0)),
        ],
        out_specs=pl.BlockSpec((BLK, H), lambda g: (g, 0)),
        out_shape=jax.ShapeDtypeStruct((E1 + E2, H), jnp.float32),
    )(hs, hd, st, te, w1a, w1b, w1c, w2, b1, b2)


CI = 2000
NVI = CI // 16
NCI = E1 // CI
CS = 160
NCS = R // CS
NVS = CS // 16


def _scatter_body(e2_hbm, vt_hbm, out_hbm, table, ibuf, gidx, rows, sem):
    sbase = _worker_id() * R

    def initb(i, carry):
        table[pl.ds(i * 16, 16)] = jnp.full((16,), -1, jnp.int32)
        return carry

    lax.fori_loop(0, R // 16, initb, 0)

    def scan_chunk(k, carry):
        pltpu.sync_copy(e2_hbm.at[pl.ds(k * CI, CI)], ibuf)

        def inner(i, c2):
            s = ibuf[pl.ds(i * 16, 16)]
            e = lax.iota(jnp.int32, 16) + (k * CI + i * 16)
            m = (s >= sbase) & (s < sbase + R)
            loc = s - sbase
            plsc.store_scatter(table, [loc], e, mask=m)
            cur = plsc.load_gather(table, [loc], mask=m)
            m2 = m & (e > cur)
            plsc.store_scatter(table, [loc], e, mask=m2)
            return c2

        lax.fori_loop(0, NVI, inner, 0)
        return carry

    lax.fori_loop(0, NCI, scan_chunk, 0)

    def write_chunk(c, carry):
        s0 = c * CS

        def mk(v, c2):
            t = table[pl.ds(s0 + v * 16, 16)]
            gslot = lax.iota(jnp.int32, 16) + (sbase + s0 + v * 16)
            gidx[pl.ds(v * 16, 16)] = jnp.where(t >= 0, t, gslot + E1)
            return c2

        lax.fori_loop(0, NVS, mk, 0)
        pltpu.async_copy(vt_hbm.at[gidx], rows, sem).wait()
        pltpu.sync_copy(rows, out_hbm.at[pl.ds(sbase + s0, CS)])
        return carry

    lax.fori_loop(0, NCS, write_chunk, 0)


_scatter_call = pl.kernel(
    _scatter_body,
    out_type=jax.ShapeDtypeStruct((E2, H), jnp.float32),
    mesh=_mesh,
    scratch_types=[
        pltpu.VMEM((R,), jnp.int32),
        pltpu.VMEM((CI,), jnp.int32),
        pltpu.VMEM((CS,), jnp.int32),
        pltpu.VMEM((CS, H), jnp.float32),
        pltpu.SemaphoreType.DMA,
    ],
    compiler_params=pltpu.CompilerParams(needs_layout_passes=False),
)


def kernel(t_e2, h, edge_index1, e1_to_e2, W1, b1, W2, b2):
    src = edge_index1[0].astype(jnp.int32)
    dst = edge_index1[1].astype(jnp.int32)
    e2i = e1_to_e2.astype(jnp.int32)
    hs, hd, st = _gather_call(h, t_e2, src, dst, e2i)
    vt = _mlp_vt(hs, hd, st, t_e2,
                 W1[0:H], W1[H:2 * H], W1[2 * H:3 * H], W2,
                 b1.reshape(1, H), b2.reshape(1, H))
    return _scatter_call(e2i, vt)


# speedup vs baseline: 3.7867x; 1.6997x over previous
import functools

import jax
import jax.numpy as jnp
from jax import lax
from jax.experimental import pallas as pl
from jax.experimental.pallas import tpu as pltpu
from jax.experimental.pallas import tpu_sc as plsc

N_NODES = 10000
E1 = 320000
E2 = 640000
H = 128

NC = 2
NS = 16
NW = NC * NS
EPW = E1 // NW
R = E2 // NW

_mesh = plsc.VectorSubcoreMesh(core_axis_name="c", subcore_axis_name="s")
_sc_params = pltpu.CompilerParams(needs_layout_passes=False)


def _worker_id():
    return lax.axis_index("s") * NC + lax.axis_index("c")


CG = 80
NCH = EPW // CG


def _gather_body(h_hbm, te_hbm, src_hbm, dst_hbm, e2_hbm,
                 hs_hbm, hd_hbm, st_hbm,
                 isrc, idst, ie2,
                 bs0, bd0, bt0, bs1, bd1, bt1,
                 sg0, sg1, so0, so1):
    base = _worker_id() * EPW
    bufs = ((bs0, bd0, bt0), (bs1, bd1, bt1))
    gsems = (sg0, sg1)
    osems = (so0, so1)

    pltpu.sync_copy(src_hbm.at[pl.ds(base, EPW)], isrc)
    pltpu.sync_copy(dst_hbm.at[pl.ds(base, EPW)], idst)
    pltpu.sync_copy(e2_hbm.at[pl.ds(base, EPW)], ie2)

    def issue_gather(k, slot):
        bs, bd, bt = bufs[slot]
        pltpu.async_copy(h_hbm.at[isrc.at[pl.ds(k * CG, CG)]], bs, gsems[slot])
        pltpu.async_copy(h_hbm.at[idst.at[pl.ds(k * CG, CG)]], bd, gsems[slot])
        pltpu.async_copy(te_hbm.at[ie2.at[pl.ds(k * CG, CG)]], bt, gsems[slot])

    def wait_gather(slot):
        bs, bd, bt = bufs[slot]
        pltpu.make_async_copy(h_hbm.at[isrc.at[pl.ds(0, CG)]], bs, gsems[slot]).wait()
        pltpu.make_async_copy(h_hbm.at[idst.at[pl.ds(0, CG)]], bd, gsems[slot]).wait()
        pltpu.make_async_copy(te_hbm.at[ie2.at[pl.ds(0, CG)]], bt, gsems[slot]).wait()

    def issue_out(k, slot):
        bs, bd, bt = bufs[slot]
        off = base + k * CG
        pltpu.async_copy(bs, hs_hbm.at[pl.ds(off, CG)], osems[slot])
        pltpu.async_copy(bd, hd_hbm.at[pl.ds(off, CG)], osems[slot])
        pltpu.async_copy(bt, st_hbm.at[pl.ds(off, CG)], osems[slot])

    def wait_out(slot):
        bs, bd, bt = bufs[slot]
        off = base
        pltpu.make_async_copy(bs, hs_hbm.at[pl.ds(off, CG)], osems[slot]).wait()
        pltpu.make_async_copy(bd, hd_hbm.at[pl.ds(off, CG)], osems[slot]).wait()
        pltpu.make_async_copy(bt, st_hbm.at[pl.ds(off, CG)], osems[slot]).wait()

    issue_gather(0, 0)
    issue_gather(1, 1)

    def ring(i, carry):
        k0 = 2 * i
        wait_gather(0)
        issue_out(k0, 0)
        wait_gather(1)
        issue_out(k0 + 1, 1)
        wait_out(0)
        issue_gather(k0 + 2, 0)
        wait_out(1)

        @pl.when(i < (NCH - 1) // 2 - 1)
        def _():
            issue_gather(k0 + 3, 1)
        return carry

    lax.fori_loop(0, (NCH - 1) // 2, ring, 0)
    wait_gather(0)
    issue_out(NCH - 1, 0)
    wait_out(0)


_gather_call = pl.kernel(
    _gather_body,
    out_type=(
        jax.ShapeDtypeStruct((E1, H), jnp.float32),
        jax.ShapeDtypeStruct((E1, H), jnp.float32),
        jax.ShapeDtypeStruct((E1, H), jnp.float32),
    ),
    mesh=_mesh,
    scratch_types=[
        pltpu.VMEM((EPW,), jnp.int32),
        pltpu.VMEM((EPW,), jnp.int32),
        pltpu.VMEM((EPW,), jnp.int32),
        pltpu.VMEM((CG, H), jnp.float32),
        pltpu.VMEM((CG, H), jnp.float32),
        pltpu.VMEM((CG, H), jnp.float32),
        pltpu.VMEM((CG, H), jnp.float32),
        pltpu.VMEM((CG, H), jnp.float32),
        pltpu.VMEM((CG, H), jnp.float32),
        pltpu.SemaphoreType.DMA,
        pltpu.SemaphoreType.DMA,
        pltpu.SemaphoreType.DMA,
        pltpu.SemaphoreType.DMA,
    ],
    compiler_params=_sc_params,
)


BLK = 3200
NB_MLP = E1 // BLK
NB_CPY = E2 // BLK
GRID = NB_MLP + NB_CPY


def _mlp_body(hs, hd, st, te, w1a, w1b, w1c, w2, b1, b2, out):
    g = pl.program_id(0)

    @pl.when(g < NB_MLP)
    def _compute():
        x = (jnp.dot(hs[...], w1a[...], preferred_element_type=jnp.float32)
             + jnp.dot(hd[...], w1b[...], preferred_element_type=jnp.float32)
             + jnp.dot(st[...], w1c[...], preferred_element_type=jnp.float32)
             + b1[...])
        hid = x * jax.nn.sigmoid(x)
        out[...] = (st[...] + b2[...]
                    + jnp.dot(hid, w2[...], preferred_element_type=jnp.float32))

    @pl.when(g >= NB_MLP)
    def _copy():
        out[...] = te[...]


def _mlp_vt(hs, hd, st, te, w1a, w1b, w1c, w2, b1, b2):
    return pl.pallas_call(
        _mlp_body,
        grid=(GRID,),
        in_specs=[
            pl.BlockSpec((BLK, H), lambda g: (jnp.minimum(g, NB_MLP - 1), 0)),
            pl.BlockSpec((BLK, H), lambda g: (jnp.minimum(g, NB_MLP - 1), 0)),
            pl.BlockSpec((BLK, H), lambda g: (jnp.minimum(g, NB_MLP - 1), 0)),
            pl.BlockSpec((BLK, H), lambda g: (jnp.maximum(g - NB_MLP, 0), 0)),
            pl.BlockSpec((H, H), lambda g: (0, 0)),
            pl.BlockSpec((H, H), lambda g: (0, 0)),
            pl.BlockSpec((H, H), lambda g: (0, 0)),
            pl.BlockSpec((H, H), lambda g: (0, 0)),
            pl.BlockSpec((1, H), lambda g: (0, 0)),
            pl.BlockSpec((1, H), lambda g: (0, 0)),
        ],
        out_specs=pl.BlockSpec((BLK, H), lambda g: (g, 0)),
        out_shape=jax.ShapeDtypeStruct((E1 + E2, H), jnp.float32),
    )(hs, hd, st, te, w1a, w1b, w1c, w2, b1, b2)


CI = 2000
NVI = CI // 16
NCI = E1 // CI
CS = 160
NCS = R // CS
NVS = CS // 16


def _scatter_body(e2_hbm, vt_hbm, out_hbm,
                  table, ib0, ib1, gi0, gi1, rw0, rw1,
                  si0, si1, sg0, sg1, so0, so1):
    sbase = _worker_id() * R
    ibufs = (ib0, ib1)
    isems = (si0, si1)
    gidxs = (gi0, gi1)
    rows = (rw0, rw1)
    gsems = (sg0, sg1)
    osems = (so0, so1)

    def initb(i, carry):
        table[pl.ds(i * 16, 16)] = jnp.full((16,), -1, jnp.int32)
        return carry

    lax.fori_loop(0, R // 16, initb, 0)

    def issue_idx(k, slot):
        pltpu.async_copy(e2_hbm.at[pl.ds(k * CI, CI)], ibufs[slot], isems[slot])

    def wait_idx(slot):
        pltpu.make_async_copy(e2_hbm.at[pl.ds(0, CI)], ibufs[slot], isems[slot]).wait()

    def scan_chunk(k, slot):
        ibuf = ibufs[slot]

        def inner(i, c2):
            s = ibuf[pl.ds(i * 16, 16)]
            e = lax.iota(jnp.int32, 16) + (k * CI + i * 16)
            m = (s >= sbase) & (s < sbase + R)
            loc = s - sbase
            plsc.store_scatter(table, [loc], e, mask=m)
            cur = plsc.load_gather(table, [loc], mask=m)
            m2 = m & (e > cur)
            plsc.store_scatter(table, [loc], e, mask=m2)
            return c2

        lax.fori_loop(0, NVI, inner, 0)

    issue_idx(0, 0)
    issue_idx(1, 1)

    def scanring(i, carry):
        k0 = 2 * i
        wait_idx(0)
        scan_chunk(k0, 0)

        @pl.when(i < NCI // 2 - 1)
        def _():
            issue_idx(k0 + 2, 0)

        wait_idx(1)
        scan_chunk(k0 + 1, 1)

        @pl.when(i < NCI // 2 - 1)
        def _():
            issue_idx(k0 + 3, 1)
        return carry

    lax.fori_loop(0, NCI // 2, scanring, 0)

    def mk_idx(c, slot):
        s0 = c * CS
        gidx = gidxs[slot]

        def mk(v, c2):
            t = table[pl.ds(s0 + v * 16, 16)]
            gslot = lax.iota(jnp.int32, 16) + (sbase + s0 + v * 16)
            gidx[pl.ds(v * 16, 16)] = jnp.where(t >= 0, t, gslot + E1)
            return c2

        lax.fori_loop(0, NVS, mk, 0)

    def issue_gather(slot):
        pltpu.async_copy(vt_hbm.at[gidxs[slot]], rows[slot], gsems[slot])

    def wait_gather(slot):
        pltpu.make_async_copy(vt_hbm.at[gidxs[slot]], rows[slot], gsems[slot]).wait()

    def issue_out(c, slot):
        pltpu.async_copy(rows[slot], out_hbm.at[pl.ds(sbase + c * CS, CS)], osems[slot])

    def wait_out(slot):
        pltpu.make_async_copy(rows[slot], out_hbm.at[pl.ds(sbase, CS)], osems[slot]).wait()

    mk_idx(0, 0)
    issue_gather(0)
    mk_idx(1, 1)
    issue_gather(1)

    def bring(i, carry):
        c0 = 2 * i
        wait_gather(0)
        issue_out(c0, 0)
        wait_gather(1)
        issue_out(c0 + 1, 1)
        wait_out(0)
        mk_idx(c0 + 2, 0)
        issue_gather(0)
        wait_out(1)

        @pl.when(i < (NCS - 1) // 2 - 1)
        def _():
            mk_idx(c0 + 3, 1)
            issue_gather(1)
        return carry

    lax.fori_loop(0, (NCS - 1) // 2, bring, 0)
    wait_gather(0)
    issue_out(NCS - 1, 0)
    wait_out(0)


_scatter_call = pl.kernel(
    _scatter_body,
    out_type=jax.ShapeDtypeStruct((E2, H), jnp.float32),
    mesh=_mesh,
    scratch_types=[
        pltpu.VMEM((R,), jnp.int32),
        pltpu.VMEM((CI,), jnp.int32),
        pltpu.VMEM((CI,), jnp.int32),
        pltpu.VMEM((CS,), jnp.int32),
        pltpu.VMEM((CS,), jnp.int32),
        pltpu.VMEM((CS, H), jnp.float32),
        pltpu.VMEM((CS, H), jnp.float32),
        pltpu.SemaphoreType.DMA,
        pltpu.SemaphoreType.DMA,
        pltpu.SemaphoreType.DMA,
        pltpu.SemaphoreType.DMA,
        pltpu.SemaphoreType.DMA,
        pltpu.SemaphoreType.DMA,
    ],
    compiler_params=_sc_params,
)


def kernel(t_e2, h, edge_index1, e1_to_e2, W1, b1, W2, b2):
    src = edge_index1[0].astype(jnp.int32)
    dst = edge_index1[1].astype(jnp.int32)
    e2i = e1_to_e2.astype(jnp.int32)
    hs, hd, st = _gather_call(h, t_e2, src, dst, e2i)
    vt = _mlp_vt(hs, hd, st, t_e2,
                 W1[0:H], W1[H:2 * H], W1[2 * H:3 * H], W2,
                 b1.reshape(1, H), b2.reshape(1, H))
    return _scatter_call(e2i, vt)


# speedup vs baseline: 4.8228x; 1.2736x over previous
import functools

import jax
import jax.numpy as jnp
from jax import lax
from jax.experimental import pallas as pl
from jax.experimental.pallas import tpu as pltpu
from jax.experimental.pallas import tpu_sc as plsc

N_NODES = 10000
E1 = 320000
E2 = 640000
H = 128

NC = 2
NS = 16
NW = NC * NS
EPW = E1 // NW
R = E2 // NW

_mesh = plsc.VectorSubcoreMesh(core_axis_name="c", subcore_axis_name="s")
_sc_params = pltpu.CompilerParams(needs_layout_passes=False)


def _worker_id():
    return lax.axis_index("s") * NC + lax.axis_index("c")


CG = 80
NCH = EPW // CG


def _gather_body(h_hbm, te_hbm, src_hbm, dst_hbm, e2_hbm,
                 hs_hbm, hd_hbm, st_hbm,
                 isrc, idst, ie2,
                 bs0, bd0, bt0, bs1, bd1, bt1,
                 sg0, sg1, so0, so1):
    base = _worker_id() * EPW
    bufs = ((bs0, bd0, bt0), (bs1, bd1, bt1))
    gsems = (sg0, sg1)
    osems = (so0, so1)

    pltpu.sync_copy(src_hbm.at[pl.ds(base, EPW)], isrc)
    pltpu.sync_copy(dst_hbm.at[pl.ds(base, EPW)], idst)
    pltpu.sync_copy(e2_hbm.at[pl.ds(base, EPW)], ie2)

    def issue_gather(k, slot):
        bs, bd, bt = bufs[slot]
        pltpu.async_copy(h_hbm.at[isrc.at[pl.ds(k * CG, CG)]], bs, gsems[slot])
        pltpu.async_copy(h_hbm.at[idst.at[pl.ds(k * CG, CG)]], bd, gsems[slot])
        pltpu.async_copy(te_hbm.at[ie2.at[pl.ds(k * CG, CG)]], bt, gsems[slot])

    def wait_gather(slot):
        bs, bd, bt = bufs[slot]
        pltpu.make_async_copy(h_hbm.at[isrc.at[pl.ds(0, CG)]], bs, gsems[slot]).wait()
        pltpu.make_async_copy(h_hbm.at[idst.at[pl.ds(0, CG)]], bd, gsems[slot]).wait()
        pltpu.make_async_copy(te_hbm.at[ie2.at[pl.ds(0, CG)]], bt, gsems[slot]).wait()

    def issue_out(k, slot):
        bs, bd, bt = bufs[slot]
        off = base + k * CG
        pltpu.async_copy(bs, hs_hbm.at[pl.ds(off, CG)], osems[slot])
        pltpu.async_copy(bd, hd_hbm.at[pl.ds(off, CG)], osems[slot])
        pltpu.async_copy(bt, st_hbm.at[pl.ds(off, CG)], osems[slot])

    def wait_out(slot):
        bs, bd, bt = bufs[slot]
        off = base
        pltpu.make_async_copy(bs, hs_hbm.at[pl.ds(off, CG)], osems[slot]).wait()
        pltpu.make_async_copy(bd, hd_hbm.at[pl.ds(off, CG)], osems[slot]).wait()
        pltpu.make_async_copy(bt, st_hbm.at[pl.ds(off, CG)], osems[slot]).wait()

    issue_gather(0, 0)
    issue_gather(1, 1)

    def ring(i, carry):
        k0 = 2 * i
        wait_gather(0)
        issue_out(k0, 0)
        wait_gather(1)
        issue_out(k0 + 1, 1)
        wait_out(0)
        issue_gather(k0 + 2, 0)
        wait_out(1)

        @pl.when(i < (NCH - 1) // 2 - 1)
        def _():
            issue_gather(k0 + 3, 1)
        return carry

    lax.fori_loop(0, (NCH - 1) // 2, ring, 0)
    wait_gather(0)
    issue_out(NCH - 1, 0)
    wait_out(0)


_gather_call = pl.kernel(
    _gather_body,
    out_type=(
        jax.ShapeDtypeStruct((E1, H), jnp.float32),
        jax.ShapeDtypeStruct((E1, H), jnp.float32),
        jax.ShapeDtypeStruct((E1, H), jnp.float32),
    ),
    mesh=_mesh,
    scratch_types=[
        pltpu.VMEM((EPW,), jnp.int32),
        pltpu.VMEM((EPW,), jnp.int32),
        pltpu.VMEM((EPW,), jnp.int32),
        pltpu.VMEM((CG, H), jnp.float32),
        pltpu.VMEM((CG, H), jnp.float32),
        pltpu.VMEM((CG, H), jnp.float32),
        pltpu.VMEM((CG, H), jnp.float32),
        pltpu.VMEM((CG, H), jnp.float32),
        pltpu.VMEM((CG, H), jnp.float32),
        pltpu.SemaphoreType.DMA,
        pltpu.SemaphoreType.DMA,
        pltpu.SemaphoreType.DMA,
        pltpu.SemaphoreType.DMA,
    ],
    compiler_params=_sc_params,
)


BLK = 3200
NB_MLP = E1 // BLK
NB_CPY = E2 // BLK
GRID = NB_MLP + NB_CPY


def _mlp_body(hs, hd, st, te, w1a, w1b, w1c, w2, b1, b2, out):
    g = pl.program_id(0)

    @pl.when(g < NB_MLP)
    def _compute():
        x = (jnp.dot(hs[...], w1a[...], preferred_element_type=jnp.float32)
             + jnp.dot(hd[...], w1b[...], preferred_element_type=jnp.float32)
             + jnp.dot(st[...], w1c[...], preferred_element_type=jnp.float32)
             + b1[...])
        hid = x * jax.nn.sigmoid(x)
        out[...] = (st[...] + b2[...]
                    + jnp.dot(hid, w2[...], preferred_element_type=jnp.float32))

    @pl.when(g >= NB_MLP)
    def _copy():
        out[...] = te[...]


def _mlp_vt(hs, hd, st, te, w1a, w1b, w1c, w2, b1, b2):
    return pl.pallas_call(
        _mlp_body,
        grid=(GRID,),
        in_specs=[
            pl.BlockSpec((BLK, H), lambda g: (jnp.minimum(g, NB_MLP - 1), 0)),
            pl.BlockSpec((BLK, H), lambda g: (jnp.minimum(g, NB_MLP - 1), 0)),
            pl.BlockSpec((BLK, H), lambda g: (jnp.minimum(g, NB_MLP - 1), 0)),
            pl.BlockSpec((BLK, H), lambda g: (jnp.maximum(g - NB_MLP, 0), 0)),
            pl.BlockSpec((H, H), lambda g: (0, 0)),
            pl.BlockSpec((H, H), lambda g: (0, 0)),
            pl.BlockSpec((H, H), lambda g: (0, 0)),
            pl.BlockSpec((H, H), lambda g: (0, 0)),
            pl.BlockSpec((1, H), lambda g: (0, 0)),
            pl.BlockSpec((1, H), lambda g: (0, 0)),
        ],
        out_specs=pl.BlockSpec((BLK, H), lambda g: (g, 0)),
        out_shape=jax.ShapeDtypeStruct((E1 + E2, H), jnp.float32),
    )(hs, hd, st, te, w1a, w1b, w1c, w2, b1, b2)


CI = 2000
NVI = CI // 16
UNR = 5
NCI = E1 // CI


def _scan_body(e2_hbm, wtab_hbm, table, ib0, ib1, si0, si1):
    sbase = _worker_id() * R
    ibufs = (ib0, ib1)
    isems = (si0, si1)

    def initb(i, carry):
        table[pl.ds(i * 16, 16)] = jnp.full((16,), -1, jnp.int32)
        return carry

    lax.fori_loop(0, R // 16, initb, 0)

    def issue_idx(k, slot):
        pltpu.async_copy(e2_hbm.at[pl.ds(k * CI, CI)], ibufs[slot], isems[slot])

    def wait_idx(slot):
        pltpu.make_async_copy(e2_hbm.at[pl.ds(0, CI)], ibufs[slot], isems[slot]).wait()

    def scan_chunk(k, slot):
        ibuf = ibufs[slot]

        def inner(i, evec):
            for u in range(UNR):
                s = ibuf[pl.ds(i * (16 * UNR) + u * 16, 16)]
                e = evec + (u * 16)
                loc = s - sbase
                m = plsc.bitcast(loc, jnp.uint32) < jnp.uint32(R)
                plsc.store_scatter(table, [loc], e, mask=m)
                cur = plsc.load_gather(table, [loc], mask=m)
                m2 = m & (e > cur)
                plsc.store_scatter(table, [loc], e, mask=m2)
            return evec + (16 * UNR)

        lax.fori_loop(0, NVI // UNR, inner,
                      lax.iota(jnp.int32, 16) + (k * CI))

    issue_idx(0, 0)
    issue_idx(1, 1)

    def scanring(i, carry):
        k0 = 2 * i
        wait_idx(0)
        scan_chunk(k0, 0)

        @pl.when(i < NCI // 2 - 1)
        def _():
            issue_idx(k0 + 2, 0)

        wait_idx(1)
        scan_chunk(k0 + 1, 1)

        @pl.when(i < NCI // 2 - 1)
        def _():
            issue_idx(k0 + 3, 1)
        return carry

    lax.fori_loop(0, NCI // 2, scanring, 0)
    pltpu.sync_copy(table, wtab_hbm.at[pl.ds(sbase, R)])


_scan_call = pl.kernel(
    _scan_body,
    out_type=jax.ShapeDtypeStruct((E2,), jnp.int32),
    mesh=_mesh,
    scratch_types=[
        pltpu.VMEM((R,), jnp.int32),
        pltpu.VMEM((CI,), jnp.int32),
        pltpu.VMEM((CI,), jnp.int32),
        pltpu.SemaphoreType.DMA,
        pltpu.SemaphoreType.DMA,
    ],
    compiler_params=_sc_params,
)


CS = 400
NCS = R // CS
NVS = CS // 16


def _write_body(wtab_hbm, vt_hbm, out_hbm,
                wb0, wb1, gi0, gi1, rw0, rw1,
                sg0, sg1, so0, so1):
    sbase = _worker_id() * R
    wbufs = (wb0, wb1)
    gidxs = (gi0, gi1)
    rows = (rw0, rw1)
    gsems = (sg0, sg1)
    osems = (so0, so1)

    def mk_idx(c, slot):
        s0 = sbase + c * CS
        pltpu.sync_copy(wtab_hbm.at[pl.ds(s0, CS)], wbufs[slot])
        gidx = gidxs[slot]
        wbuf = wbufs[slot]

        def mk(v, c2):
            t = wbuf[pl.ds(v * 16, 16)]
            gslot = lax.iota(jnp.int32, 16) + (s0 + v * 16)
            gidx[pl.ds(v * 16, 16)] = jnp.where(t >= 0, t, gslot + E1)
            return c2

        lax.fori_loop(0, NVS, mk, 0)

    def issue_gather(slot):
        pltpu.async_copy(vt_hbm.at[gidxs[slot]], rows[slot], gsems[slot])

    def wait_gather(slot):
        pltpu.make_async_copy(vt_hbm.at[gidxs[slot]], rows[slot], gsems[slot]).wait()

    def issue_out(c, slot):
        pltpu.async_copy(rows[slot], out_hbm.at[pl.ds(sbase + c * CS, CS)], osems[slot])

    def wait_out(slot):
        pltpu.make_async_copy(rows[slot], out_hbm.at[pl.ds(sbase, CS)], osems[slot]).wait()

    mk_idx(0, 0)
    issue_gather(0)
    mk_idx(1, 1)
    issue_gather(1)

    def bring(i, carry):
        c0 = 2 * i
        wait_gather(0)
        issue_out(c0, 0)
        wait_gather(1)
        issue_out(c0 + 1, 1)
        wait_out(0)

        @pl.when(i < NCS // 2 - 1)
        def _():
            mk_idx(c0 + 2, 0)
            issue_gather(0)

        wait_out(1)

        @pl.when(i < NCS // 2 - 1)
        def _():
            mk_idx(c0 + 3, 1)
            issue_gather(1)
        return carry

    lax.fori_loop(0, NCS // 2, bring, 0)


_write_call = pl.kernel(
    _write_body,
    out_type=jax.ShapeDtypeStruct((E2, H), jnp.float32),
    mesh=_mesh,
    scratch_types=[
        pltpu.VMEM((CS,), jnp.int32),
        pltpu.VMEM((CS,), jnp.int32),
        pltpu.VMEM((CS,), jnp.int32),
        pltpu.VMEM((CS,), jnp.int32),
        pltpu.VMEM((CS, H), jnp.float32),
        pltpu.VMEM((CS, H), jnp.float32),
        pltpu.SemaphoreType.DMA,
        pltpu.SemaphoreType.DMA,
        pltpu.SemaphoreType.DMA,
        pltpu.SemaphoreType.DMA,
    ],
    compiler_params=_sc_params,
)


def kernel(t_e2, h, edge_index1, e1_to_e2, W1, b1, W2, b2):
    src = edge_index1[0].astype(jnp.int32)
    dst = edge_index1[1].astype(jnp.int32)
    e2i = e1_to_e2.astype(jnp.int32)
    hs, hd, st = _gather_call(h, t_e2, src, dst, e2i)
    wtab = _scan_call(e2i)
    vt = _mlp_vt(hs, hd, st, t_e2,
                 W1[0:H], W1[H:2 * H], W1[2 * H:3 * H], W2,
                 b1.reshape(1, H), b2.reshape(1, H))
    return _write_call(wtab, vt)


# speedup vs baseline: 5.1914x; 1.0764x over previous
import functools

import jax
import jax.numpy as jnp
from jax import lax
from jax.experimental import pallas as pl
from jax.experimental.pallas import tpu as pltpu
from jax.experimental.pallas import tpu_sc as plsc

N_NODES = 10000
E1 = 320000
E2 = 640000
H = 128

NC = 2
NS = 16
NW = NC * NS
EPW = E1 // NW
R = E2 // NW

_mesh = plsc.VectorSubcoreMesh(core_axis_name="c", subcore_axis_name="s")
_sc_params = pltpu.CompilerParams(needs_layout_passes=False)


def _worker_id():
    return lax.axis_index("s") * NC + lax.axis_index("c")


CG = 80
NCH = EPW // CG


def _gather_body(h_hbm, te_hbm, src_hbm, dst_hbm, e2_hbm,
                 hs_hbm, hd_hbm, st_hbm,
                 isrc, idst, ie2,
                 bs0, bd0, bt0, bs1, bd1, bt1,
                 sg0, sg1, so0, so1):
    base = _worker_id() * EPW
    bufs = ((bs0, bd0, bt0), (bs1, bd1, bt1))
    gsems = (sg0, sg1)
    osems = (so0, so1)

    pltpu.sync_copy(src_hbm.at[pl.ds(base, EPW)], isrc)
    pltpu.sync_copy(dst_hbm.at[pl.ds(base, EPW)], idst)
    pltpu.sync_copy(e2_hbm.at[pl.ds(base, EPW)], ie2)

    def issue_gather(k, slot):
        bs, bd, bt = bufs[slot]
        pltpu.async_copy(h_hbm.at[isrc.at[pl.ds(k * CG, CG)]], bs, gsems[slot])
        pltpu.async_copy(h_hbm.at[idst.at[pl.ds(k * CG, CG)]], bd, gsems[slot])
        pltpu.async_copy(te_hbm.at[ie2.at[pl.ds(k * CG, CG)]], bt, gsems[slot])

    def wait_gather(slot):
        bs, bd, bt = bufs[slot]
        pltpu.make_async_copy(h_hbm.at[isrc.at[pl.ds(0, CG)]], bs, gsems[slot]).wait()
        pltpu.make_async_copy(h_hbm.at[idst.at[pl.ds(0, CG)]], bd, gsems[slot]).wait()
        pltpu.make_async_copy(te_hbm.at[ie2.at[pl.ds(0, CG)]], bt, gsems[slot]).wait()

    def issue_out(k, slot):
        bs, bd, bt = bufs[slot]
        off = base + k * CG
        pltpu.async_copy(bs, hs_hbm.at[pl.ds(off, CG)], osems[slot])
        pltpu.async_copy(bd, hd_hbm.at[pl.ds(off, CG)], osems[slot])
        pltpu.async_copy(bt, st_hbm.at[pl.ds(off, CG)], osems[slot])

    def wait_out(slot):
        bs, bd, bt = bufs[slot]
        off = base
        pltpu.make_async_copy(bs, hs_hbm.at[pl.ds(off, CG)], osems[slot]).wait()
        pltpu.make_async_copy(bd, hd_hbm.at[pl.ds(off, CG)], osems[slot]).wait()
        pltpu.make_async_copy(bt, st_hbm.at[pl.ds(off, CG)], osems[slot]).wait()

    issue_gather(0, 0)
    issue_gather(1, 1)

    def ring(i, carry):
        k0 = 2 * i
        wait_gather(0)
        issue_out(k0, 0)
        wait_gather(1)
        issue_out(k0 + 1, 1)
        wait_out(0)
        issue_gather(k0 + 2, 0)
        wait_out(1)

        @pl.when(i < (NCH - 1) // 2 - 1)
        def _():
            issue_gather(k0 + 3, 1)
        return carry

    lax.fori_loop(0, (NCH - 1) // 2, ring, 0)
    wait_gather(0)
    issue_out(NCH - 1, 0)
    wait_out(0)


_gather_call = pl.kernel(
    _gather_body,
    out_type=(
        jax.ShapeDtypeStruct((E1, H), jnp.float32),
        jax.ShapeDtypeStruct((E1, H), jnp.float32),
        jax.ShapeDtypeStruct((E1, H), jnp.float32),
    ),
    mesh=_mesh,
    scratch_types=[
        pltpu.VMEM((EPW,), jnp.int32),
        pltpu.VMEM((EPW,), jnp.int32),
        pltpu.VMEM((EPW,), jnp.int32),
        pltpu.VMEM((CG, H), jnp.float32),
        pltpu.VMEM((CG, H), jnp.float32),
        pltpu.VMEM((CG, H), jnp.float32),
        pltpu.VMEM((CG, H), jnp.float32),
        pltpu.VMEM((CG, H), jnp.float32),
        pltpu.VMEM((CG, H), jnp.float32),
        pltpu.SemaphoreType.DMA,
        pltpu.SemaphoreType.DMA,
        pltpu.SemaphoreType.DMA,
        pltpu.SemaphoreType.DMA,
    ],
    compiler_params=_sc_params,
)


BLK = 3200
NB_MLP = E1 // BLK


def _mlp_body(hs, hd, st, w1a, w1b, w1c, w2, b1, b2, out):
    x = (jnp.dot(hs[...], w1a[...], preferred_element_type=jnp.float32)
         + jnp.dot(hd[...], w1b[...], preferred_element_type=jnp.float32)
         + jnp.dot(st[...], w1c[...], preferred_element_type=jnp.float32)
         + b1[...])
    hid = x * jax.nn.sigmoid(x)
    out[...] = (st[...] + b2[...]
                + jnp.dot(hid, w2[...], preferred_element_type=jnp.float32))


def _mlp_v(hs, hd, st, w1a, w1b, w1c, w2, b1, b2):
    return pl.pallas_call(
        _mlp_body,
        grid=(NB_MLP,),
        in_specs=[
            pl.BlockSpec((BLK, H), lambda g: (g, 0)),
            pl.BlockSpec((BLK, H), lambda g: (g, 0)),
            pl.BlockSpec((BLK, H), lambda g: (g, 0)),
            pl.BlockSpec((H, H), lambda g: (0, 0)),
            pl.BlockSpec((H, H), lambda g: (0, 0)),
            pl.BlockSpec((H, H), lambda g: (0, 0)),
            pl.BlockSpec((H, H), lambda g: (0, 0)),
            pl.BlockSpec((1, H), lambda g: (0, 0)),
            pl.BlockSpec((1, H), lambda g: (0, 0)),
        ],
        out_specs=pl.BlockSpec((BLK, H), lambda g: (g, 0)),
        out_shape=jax.ShapeDtypeStruct((E1, H), jnp.float32),
    )(hs, hd, st, w1a, w1b, w1c, w2, b1, b2)


CI = 2000
NVI = CI // 16
UNR = 5
NCI = E1 // CI


def _scan_body(e2_hbm, wtab_hbm, table, ib0, ib1, si0, si1):
    sbase = _worker_id() * R
    ibufs = (ib0, ib1)
    isems = (si0, si1)

    def initb(i, carry):
        table[pl.ds(i * 16, 16)] = jnp.full((16,), -1, jnp.int32)
        return carry

    lax.fori_loop(0, R // 16, initb, 0)

    def issue_idx(k, slot):
        pltpu.async_copy(e2_hbm.at[pl.ds(k * CI, CI)], ibufs[slot], isems[slot])

    def wait_idx(slot):
        pltpu.make_async_copy(e2_hbm.at[pl.ds(0, CI)], ibufs[slot], isems[slot]).wait()

    def scan_chunk(k, slot):
        ibuf = ibufs[slot]

        def inner(i, evec):
            for u in range(UNR):
                s = ibuf[pl.ds(i * (16 * UNR) + u * 16, 16)]
                e = evec + (u * 16)
                loc = s - sbase
                m = plsc.bitcast(loc, jnp.uint32) < jnp.uint32(R)
                plsc.store_scatter(table, [loc], e, mask=m)
                cur = plsc.load_gather(table, [loc], mask=m)
                m2 = m & (e > cur)
                plsc.store_scatter(table, [loc], e, mask=m2)
            return evec + (16 * UNR)

        lax.fori_loop(0, NVI // UNR, inner,
                      lax.iota(jnp.int32, 16) + (k * CI))

    issue_idx(0, 0)
    issue_idx(1, 1)

    def scanring(i, carry):
        k0 = 2 * i
        wait_idx(0)
        scan_chunk(k0, 0)

        @pl.when(i < NCI // 2 - 1)
        def _():
            issue_idx(k0 + 2, 0)

        wait_idx(1)
        scan_chunk(k0 + 1, 1)

        @pl.when(i < NCI // 2 - 1)
        def _():
            issue_idx(k0 + 3, 1)
        return carry

    lax.fori_loop(0, NCI // 2, scanring, 0)
    pltpu.sync_copy(table, wtab_hbm.at[pl.ds(sbase, R)])


_scan_call = pl.kernel(
    _scan_body,
    out_type=jax.ShapeDtypeStruct((E2,), jnp.int32),
    mesh=_mesh,
    scratch_types=[
        pltpu.VMEM((R,), jnp.int32),
        pltpu.VMEM((CI,), jnp.int32),
        pltpu.VMEM((CI,), jnp.int32),
        pltpu.SemaphoreType.DMA,
        pltpu.SemaphoreType.DMA,
    ],
    compiler_params=_sc_params,
)


GCH = 128
NCW = (R + GCH - 1) // GCH + 1


def _winner_body(wtab_hbm, v_hbm, o_hbm,
                 tbuf, ws2, we2, rb0, rb1,
                 sg0, sg1, ss0, ss1):
    sbase = _worker_id() * R
    rbufs = (rb0, rb1)
    gsems = (sg0, sg1)
    ssems = (ss0, ss1)

    pltpu.sync_copy(wtab_hbm.at[pl.ds(sbase, R)], tbuf)

    def comp(i, nwv):
        t = tbuf[pl.ds(i * 16, 16)]
        m = t >= 0
        slots = lax.iota(jnp.int32, 16) + (sbase + i * 16)
        cnt = plsc.cumsum(jnp.where(m, 1, 0))
        pos = nwv + cnt - 1
        plsc.store_scatter(ws2, [pos >> 7, pos & 127], slots, mask=m)
        plsc.store_scatter(we2, [pos >> 7, pos & 127], t, mask=m)
        return nwv + plsc.all_reduce_population_count(m)

    nwv = lax.fori_loop(0, R // 16, comp,
                        jnp.zeros((16,), jnp.int32))

    z = jnp.zeros((16,), jnp.int32)
    w0s = plsc.load_gather(ws2, [z, z])
    w0e = plsc.load_gather(we2, [z, z])
    end = ((nwv + 127) >> 7) << 7
    for j in range(GCH // 16):
        pos = nwv + lax.iota(jnp.int32, 16) + (j * 16)
        mf = pos < end
        plsc.store_scatter(ws2, [pos >> 7, pos & 127], w0s, mask=mf)
        plsc.store_scatter(we2, [pos >> 7, pos & 127], w0e, mask=mf)

    nw = jnp.max(nwv)
    nch = (nw + GCH - 1) // GCH

    def issue_gather(c, slot):
        pltpu.async_copy(v_hbm.at[we2.at[c]], rbufs[slot], gsems[slot])

    def wait_gather(slot):
        pltpu.make_async_copy(v_hbm.at[we2.at[0]], rbufs[slot], gsems[slot]).wait()

    def issue_scat(c, slot):
        pltpu.async_copy(rbufs[slot], o_hbm.at[ws2.at[c]], ssems[slot])

    def wait_scat(slot):
        pltpu.make_async_copy(rbufs[slot], o_hbm.at[ws2.at[0]], ssems[slot]).wait()

    def pair(i, carry):
        c0 = 2 * i
        c1 = c0 + 1
        issue_gather(c0, 0)

        @pl.when(c1 < nch)
        def _():
            issue_gather(c1, 1)

        wait_gather(0)
        issue_scat(c0, 0)

        @pl.when(c1 < nch)
        def _():
            wait_gather(1)
            issue_scat(c1, 1)

        wait_scat(0)

        @pl.when(c1 < nch)
        def _():
            wait_scat(1)
        return carry

    lax.fori_loop(0, (nch + 1) // 2, pair, 0)


_winner_call = pl.kernel(
    _winner_body,
    out_type=(),
    mesh=_mesh,
    scratch_types=[
        pltpu.VMEM((R,), jnp.int32),
        pltpu.VMEM((NCW, GCH), jnp.int32),
        pltpu.VMEM((NCW, GCH), jnp.int32),
        pltpu.VMEM((GCH, H), jnp.float32),
        pltpu.VMEM((GCH, H), jnp.float32),
        pltpu.SemaphoreType.DMA,
        pltpu.SemaphoreType.DMA,
        pltpu.SemaphoreType.DMA,
        pltpu.SemaphoreType.DMA,
    ],
    compiler_params=_sc_params,
)


def kernel(t_e2, h, edge_index1, e1_to_e2, W1, b1, W2, b2):
    src = edge_index1[0].astype(jnp.int32)
    dst = edge_index1[1].astype(jnp.int32)
    e2i = e1_to_e2.astype(jnp.int32)
    o_ref = jax.new_ref(t_e2)
    hs, hd, st = _gather_call(h, t_e2, src, dst, e2i)
    wtab = _scan_call(e2i)
    v = _mlp_v(hs, hd, st,
               W1[0:H], W1[H:2 * H], W1[2 * H:3 * H], W2,
               b1.reshape(1, H), b2.reshape(1, H))
    _winner_call(wtab, v, o_ref)
    return jax.freeze(o_ref)
